# baseline (speedup 1.0000x reference)
import functools

import jax
import jax.numpy as jnp
from jax import lax
from jax.experimental import pallas as pl
from jax.experimental.pallas import tpu as pltpu
from jax.experimental.pallas import tpu_sc as plsc

N_NODES = 10000
N_EDGES = 320000
IN_CH, HID, OUT_CH = 128, 16, 128

_NC, _NS = 2, 16
_NW = _NC * _NS
_CHUNK = 128
_CPT = -(-(-(-N_EDGES // (_NW * _CHUNK))) // 8) * 8
_E_PAD = _NW * _CPT * _CHUNK
_ROWS = _E_PAD // _CHUNK
_TRASH = N_NODES
_NPAD = 10240
_D = 8
_C0 = 80
_C1 = 2 * _CPT - _C0
_CMAX = max(_C0, _C1)
_RPT = _NPAD // _NS

_mesh = plsc.VectorSubcoreMesh(
    core_axis_name="c", subcore_axis_name="s", num_cores=_NC, num_subcores=_NS
)


@functools.partial(
    pl.kernel,
    out_type=jax.ShapeDtypeStruct((_NC, _NPAD), jnp.float32),
    mesh=_mesh,
    scratch_types=[
        pltpu.VMEM((_CPT, _CHUNK), jnp.int32),
        pltpu.VMEM((_CHUNK,), jnp.float32),
        pltpu.VMEM((_RPT,), jnp.float32),
        pltpu.VMEM_SHARED((_NPAD,), jnp.float32),
    ],
)
def _deg_kernel(dst_hbm, out_hbm, didx, ones, zbuf, deg_sh):
    cid = lax.axis_index("c")
    sid = lax.axis_index("s")
    tid = cid * _NS + sid

    def _init(i, _):
        ones[pl.ds(i * 16, 16)] = jnp.ones((16,), jnp.float32)
        zbuf[pl.ds(i * 16, 16)] = jnp.zeros((16,), jnp.float32)
        return 0

    lax.fori_loop(0, _CHUNK // 16, _init, 0)

    def _zinit(i, _):
        zbuf[pl.ds(i * 16, 16)] = jnp.zeros((16,), jnp.float32)
        return 0

    lax.fori_loop(0, _RPT // 16, _zinit, 0)

    pltpu.sync_copy(zbuf, deg_sh.at[pl.ds(sid * _RPT, _RPT)])
    pltpu.sync_copy(dst_hbm.at[pl.ds(tid * _CPT, _CPT)], didx)
    plsc.subcore_barrier()

    def _scat(j, _):
        pltpu.sync_copy(ones, deg_sh.at[didx.at[j]], add=True)
        return 0

    lax.fori_loop(0, _CPT, _scat, 0)
    plsc.subcore_barrier()

    pltpu.sync_copy(deg_sh.at[pl.ds(sid * _RPT, _RPT)], zbuf)
    pltpu.sync_copy(zbuf, out_hbm.at[cid, pl.ds(sid * _RPT, _RPT)])


@functools.partial(
    pl.kernel,
    out_type=jax.ShapeDtypeStruct((_NC, _NPAD, HID), jnp.float32),
    mesh=_mesh,
    compiler_params=pltpu.CompilerParams(use_tc_tiling_on_sc=False),
    scratch_types=[
        pltpu.VMEM((_CMAX, _CHUNK), jnp.int32),
        pltpu.VMEM((_CMAX, _CHUNK), jnp.int32),
        pltpu.VMEM((_D, _CHUNK, HID), jnp.float32),
        pltpu.VMEM((_RPT, HID), jnp.float32),
        pltpu.VMEM_SHARED((_NPAD, HID), jnp.float32),
        pltpu.SemaphoreType.DMA((_D,)),
        pltpu.SemaphoreType.DMA((_D,)),
    ],
)
def _agg_kernel(src_hbm, dst_hbm, g_hbm, out_hbm, sidx, didx, rows, zbuf, agg_sh,
                gsem, ssem):
    cid = lax.axis_index("c")
    sid = lax.axis_index("s")
    my_cnt = jnp.where(cid == 0, _C0, _C1)
    base_row = jnp.where(cid == 0, sid * _C0, _NS * _C0 + sid * _C1)

    def _zinit(i, _):
        zbuf[i, :] = jnp.zeros((HID,), jnp.float32)
        return 0

    lax.fori_loop(0, _RPT, _zinit, 0)

    pltpu.sync_copy(zbuf, agg_sh.at[pl.ds(sid * _RPT, _RPT)])

    @pl.when(cid == 0)
    def _():
        pltpu.sync_copy(src_hbm.at[pl.ds(base_row, _C0)], sidx.at[pl.ds(0, _C0)])
        pltpu.sync_copy(dst_hbm.at[pl.ds(base_row, _C0)], didx.at[pl.ds(0, _C0)])

    @pl.when(cid == 1)
    def _():
        pltpu.sync_copy(src_hbm.at[pl.ds(base_row, _C1)], sidx.at[pl.ds(0, _C1)])
        pltpu.sync_copy(dst_hbm.at[pl.ds(base_row, _C1)], didx.at[pl.ds(0, _C1)])

    plsc.subcore_barrier()

    def _group(g, _):
        base = g * _D
        for b in range(_D):
            j = base + b

            @pl.when(g > 0)
            def _():
                pltpu.make_async_copy(
                    rows.at[b], agg_sh.at[didx.at[j]], ssem.at[b]
                ).wait()

            pltpu.async_copy(g_hbm.at[sidx.at[j]], rows.at[b], gsem.at[b])
        for b in range(_D):
            j = base + b
            pltpu.make_async_copy(g_hbm.at[sidx.at[j]], rows.at[b], gsem.at[b]).wait()
            pltpu.make_async_copy(rows.at[b], agg_sh.at[didx.at[j]], ssem.at[b]).start(
                add=True
            )
        return 0

    lax.fori_loop(0, my_cnt // _D, _group, 0)
    for b in range(_D):
        pltpu.make_async_copy(rows.at[b], agg_sh.at[didx.at[b]], ssem.at[b]).wait()
    plsc.subcore_barrier()

    pltpu.sync_copy(agg_sh.at[pl.ds(sid * _RPT, _RPT)], zbuf)
    pltpu.sync_copy(zbuf, out_hbm.at[cid, pl.ds(sid * _RPT, _RPT)])


_BLK = 1000
_GRID = N_NODES // _BLK


def _tc1_body(x_ref, w1_ref, degp_ref, g1_ref, dinv_ref):
    deg = 1.0 + degp_ref[:, 0] + degp_ref[:, 1]
    dinv = lax.rsqrt(deg)[:, None]
    h = jnp.dot(x_ref[...], w1_ref[...], preferred_element_type=jnp.float32)
    g1_ref[...] = h * dinv
    dinv_ref[...] = dinv


def _tc1(x, w1, degp):
    return pl.pallas_call(
        _tc1_body,
        grid=(_GRID,),
        in_specs=[
            pl.BlockSpec((_BLK, IN_CH), lambda i: (i, 0)),
            pl.BlockSpec((IN_CH, HID), lambda i: (0, 0)),
            pl.BlockSpec((_BLK, 2), lambda i: (i, 0)),
        ],
        out_specs=[
            pl.BlockSpec((_BLK, HID), lambda i: (i, 0)),
            pl.BlockSpec((_BLK, 1), lambda i: (i, 0)),
        ],
        out_shape=[
            jax.ShapeDtypeStruct((N_NODES, HID), jnp.float32),
            jax.ShapeDtypeStruct((N_NODES, 1), jnp.float32),
        ],
    )(x, w1, degp)


def _tc2_body(aggp_ref, g1_ref, dinv_ref, b1_ref, g2_ref):
    agg = aggp_ref[0] + aggp_ref[1] + g1_ref[...]
    dinv = dinv_ref[...]
    h1 = jnp.maximum(dinv * agg + b1_ref[...], 0.0)
    g2_ref[...] = dinv * h1


def _tc2(aggp, g1, dinv, b1):
    return pl.pallas_call(
        _tc2_body,
        grid=(_GRID,),
        in_specs=[
            pl.BlockSpec((2, _BLK, HID), lambda i: (0, i, 0)),
            pl.BlockSpec((_BLK, HID), lambda i: (i, 0)),
            pl.BlockSpec((_BLK, 1), lambda i: (i, 0)),
            pl.BlockSpec((1, HID), lambda i: (0, 0)),
        ],
        out_specs=pl.BlockSpec((_BLK, HID), lambda i: (i, 0)),
        out_shape=jax.ShapeDtypeStruct((N_NODES, HID), jnp.float32),
    )(aggp, g1, dinv, b1)


def _tc3_body(aggp_ref, g2_ref, dinv_ref, w2_ref, b2_ref, out_ref):
    p = dinv_ref[...] * (aggp_ref[0] + aggp_ref[1] + g2_ref[...])
    out_ref[...] = (
        jnp.dot(p, w2_ref[...], preferred_element_type=jnp.float32) + b2_ref[...]
    )


def _tc3(aggp, g2, dinv, w2, b2):
    return pl.pallas_call(
        _tc3_body,
        grid=(_GRID,),
        in_specs=[
            pl.BlockSpec((2, _BLK, HID), lambda i: (0, i, 0)),
            pl.BlockSpec((_BLK, HID), lambda i: (i, 0)),
            pl.BlockSpec((_BLK, 1), lambda i: (i, 0)),
            pl.BlockSpec((HID, OUT_CH), lambda i: (0, 0)),
            pl.BlockSpec((1, OUT_CH), lambda i: (0, 0)),
        ],
        out_specs=pl.BlockSpec((_BLK, OUT_CH), lambda i: (i, 0)),
        out_shape=jax.ShapeDtypeStruct((N_NODES, OUT_CH), jnp.float32),
    )(aggp, g2, dinv, w2, b2)


def kernel(x, edge_index, W1, b1, W2, b2):
    src = edge_index[0].astype(jnp.int32)
    dst = edge_index[1].astype(jnp.int32)
    pad = _E_PAD - N_EDGES
    src_r = jnp.concatenate([src, jnp.zeros((pad,), jnp.int32)]).reshape(_ROWS, _CHUNK)
    trash = _TRASH + jnp.arange(pad, dtype=jnp.int32) % (_NPAD - _TRASH)
    dst_r = jnp.concatenate([dst, trash]).reshape(_ROWS, _CHUNK)
    degp = _deg_kernel(dst_r)
    g1, dinv = _tc1(x, W1, degp.T)
    agg1 = _agg_kernel(src_r, dst_r, g1)
    g2 = _tc2(agg1, g1, dinv, b1.reshape(1, HID))
    agg2 = _agg_kernel(src_r, dst_r, g2)
    return _tc3(agg2, g2, dinv, W2, b2.reshape(1, OUT_CH))


# speedup vs baseline: 1.0289x; 1.0289x over previous
import functools

import jax
import jax.numpy as jnp
from jax import lax
from jax.experimental import pallas as pl
from jax.experimental.pallas import tpu as pltpu
from jax.experimental.pallas import tpu_sc as plsc

N_NODES = 10000
N_EDGES = 320000
IN_CH, HID, OUT_CH = 128, 16, 128

_NC, _NS = 2, 16
_NW = _NC * _NS
_CHUNK = 128
_CPT = -(-(-(-N_EDGES // (_NW * _CHUNK))) // 8) * 8
_E_PAD = _NW * _CPT * _CHUNK
_ROWS = _E_PAD // _CHUNK
_TRASH = N_NODES
_NPAD = 10240
_D = 8
_C0 = 96
_C1 = 2 * _CPT - _C0
_CMAX = max(_C0, _C1)
_RPT = _NPAD // _NS

_mesh = plsc.VectorSubcoreMesh(
    core_axis_name="c", subcore_axis_name="s", num_cores=_NC, num_subcores=_NS
)


@functools.partial(
    pl.kernel,
    out_type=jax.ShapeDtypeStruct((_NC, _NPAD), jnp.float32),
    mesh=_mesh,
    scratch_types=[
        pltpu.VMEM((_CPT, _CHUNK), jnp.int32),
        pltpu.VMEM((_CHUNK,), jnp.float32),
        pltpu.VMEM((_RPT,), jnp.float32),
        pltpu.VMEM_SHARED((_NPAD,), jnp.float32),
    ],
)
def _deg_kernel(dst_hbm, out_hbm, didx, ones, zbuf, deg_sh):
    cid = lax.axis_index("c")
    sid = lax.axis_index("s")
    tid = cid * _NS + sid

    def _init(i, _):
        ones[pl.ds(i * 16, 16)] = jnp.ones((16,), jnp.float32)
        zbuf[pl.ds(i * 16, 16)] = jnp.zeros((16,), jnp.float32)
        return 0

    lax.fori_loop(0, _CHUNK // 16, _init, 0)

    def _zinit(i, _):
        zbuf[pl.ds(i * 16, 16)] = jnp.zeros((16,), jnp.float32)
        return 0

    lax.fori_loop(0, _RPT // 16, _zinit, 0)

    pltpu.sync_copy(zbuf, deg_sh.at[pl.ds(sid * _RPT, _RPT)])
    pltpu.sync_copy(dst_hbm.at[pl.ds(tid * _CPT, _CPT)], didx)
    plsc.subcore_barrier()

    def _scat(j, _):
        pltpu.sync_copy(ones, deg_sh.at[didx.at[j]], add=True)
        return 0

    lax.fori_loop(0, _CPT, _scat, 0)
    plsc.subcore_barrier()

    pltpu.sync_copy(deg_sh.at[pl.ds(sid * _RPT, _RPT)], zbuf)
    pltpu.sync_copy(zbuf, out_hbm.at[cid, pl.ds(sid * _RPT, _RPT)])


@functools.partial(
    pl.kernel,
    out_type=jax.ShapeDtypeStruct((_NC, _NPAD, HID), jnp.float32),
    mesh=_mesh,
    compiler_params=pltpu.CompilerParams(use_tc_tiling_on_sc=False),
    scratch_types=[
        pltpu.VMEM((_CMAX, _CHUNK), jnp.int32),
        pltpu.VMEM((_CMAX, _CHUNK), jnp.int32),
        pltpu.VMEM((_D, _CHUNK, HID), jnp.float32),
        pltpu.VMEM((_RPT, HID), jnp.float32),
        pltpu.VMEM_SHARED((_NPAD, HID), jnp.float32),
        pltpu.SemaphoreType.DMA((_D,)),
        pltpu.SemaphoreType.DMA((_D,)),
    ],
)
def _agg_kernel(src_hbm, dst_hbm, g_hbm, out_hbm, sidx, didx, rows, zbuf, agg_sh,
                gsem, ssem):
    cid = lax.axis_index("c")
    sid = lax.axis_index("s")
    my_cnt = jnp.where(cid == 0, _C0, _C1)
    base_row = jnp.where(cid == 0, sid * _C0, _NS * _C0 + sid * _C1)

    def _zinit(i, _):
        zbuf[i, :] = jnp.zeros((HID,), jnp.float32)
        return 0

    lax.fori_loop(0, _RPT, _zinit, 0)

    pltpu.sync_copy(zbuf, agg_sh.at[pl.ds(sid * _RPT, _RPT)])

    @pl.when(cid == 0)
    def _():
        pltpu.sync_copy(src_hbm.at[pl.ds(base_row, _C0)], sidx.at[pl.ds(0, _C0)])
        pltpu.sync_copy(dst_hbm.at[pl.ds(base_row, _C0)], didx.at[pl.ds(0, _C0)])

    @pl.when(cid == 1)
    def _():
        pltpu.sync_copy(src_hbm.at[pl.ds(base_row, _C1)], sidx.at[pl.ds(0, _C1)])
        pltpu.sync_copy(dst_hbm.at[pl.ds(base_row, _C1)], didx.at[pl.ds(0, _C1)])

    plsc.subcore_barrier()

    def _group(g, _):
        base = g * _D
        for b in range(_D):
            j = base + b

            @pl.when(g > 0)
            def _():
                pltpu.make_async_copy(
                    rows.at[b], agg_sh.at[didx.at[j]], ssem.at[b]
                ).wait()

            pltpu.async_copy(g_hbm.at[sidx.at[j]], rows.at[b], gsem.at[b])
        for b in range(_D):
            j = base + b
            pltpu.make_async_copy(g_hbm.at[sidx.at[j]], rows.at[b], gsem.at[b]).wait()
            pltpu.make_async_copy(rows.at[b], agg_sh.at[didx.at[j]], ssem.at[b]).start(
                add=True
            )
        return 0

    lax.fori_loop(0, my_cnt // _D, _group, 0)
    for b in range(_D):
        pltpu.make_async_copy(rows.at[b], agg_sh.at[didx.at[b]], ssem.at[b]).wait()
    plsc.subcore_barrier()

    pltpu.sync_copy(agg_sh.at[pl.ds(sid * _RPT, _RPT)], zbuf)
    pltpu.sync_copy(zbuf, out_hbm.at[cid, pl.ds(sid * _RPT, _RPT)])


_BLK = 1000
_GRID = N_NODES // _BLK


def _tc1_body(x_ref, w1_ref, degp_ref, g1_ref, dinv_ref):
    deg = 1.0 + degp_ref[:, 0] + degp_ref[:, 1]
    dinv = lax.rsqrt(deg)[:, None]
    h = jnp.dot(x_ref[...], w1_ref[...], preferred_element_type=jnp.float32)
    g1_ref[...] = h * dinv
    dinv_ref[...] = dinv


def _tc1(x, w1, degp):
    return pl.pallas_call(
        _tc1_body,
        grid=(_GRID,),
        in_specs=[
            pl.BlockSpec((_BLK, IN_CH), lambda i: (i, 0)),
            pl.BlockSpec((IN_CH, HID), lambda i: (0, 0)),
            pl.BlockSpec((_BLK, 2), lambda i: (i, 0)),
        ],
        out_specs=[
            pl.BlockSpec((_BLK, HID), lambda i: (i, 0)),
            pl.BlockSpec((_BLK, 1), lambda i: (i, 0)),
        ],
        out_shape=[
            jax.ShapeDtypeStruct((N_NODES, HID), jnp.float32),
            jax.ShapeDtypeStruct((N_NODES, 1), jnp.float32),
        ],
    )(x, w1, degp)


def _tc2_body(aggp_ref, g1_ref, dinv_ref, b1_ref, g2_ref):
    agg = aggp_ref[0] + aggp_ref[1] + g1_ref[...]
    dinv = dinv_ref[...]
    h1 = jnp.maximum(dinv * agg + b1_ref[...], 0.0)
    g2_ref[...] = dinv * h1


def _tc2(aggp, g1, dinv, b1):
    return pl.pallas_call(
        _tc2_body,
        grid=(_GRID,),
        in_specs=[
            pl.BlockSpec((2, _BLK, HID), lambda i: (0, i, 0)),
            pl.BlockSpec((_BLK, HID), lambda i: (i, 0)),
            pl.BlockSpec((_BLK, 1), lambda i: (i, 0)),
            pl.BlockSpec((1, HID), lambda i: (0, 0)),
        ],
        out_specs=pl.BlockSpec((_BLK, HID), lambda i: (i, 0)),
        out_shape=jax.ShapeDtypeStruct((N_NODES, HID), jnp.float32),
    )(aggp, g1, dinv, b1)


def _tc3_body(aggp_ref, g2_ref, dinv_ref, w2_ref, b2_ref, out_ref):
    p = dinv_ref[...] * (aggp_ref[0] + aggp_ref[1] + g2_ref[...])
    out_ref[...] = (
        jnp.dot(p, w2_ref[...], preferred_element_type=jnp.float32) + b2_ref[...]
    )


def _tc3(aggp, g2, dinv, w2, b2):
    return pl.pallas_call(
        _tc3_body,
        grid=(_GRID,),
        in_specs=[
            pl.BlockSpec((2, _BLK, HID), lambda i: (0, i, 0)),
            pl.BlockSpec((_BLK, HID), lambda i: (i, 0)),
            pl.BlockSpec((_BLK, 1), lambda i: (i, 0)),
            pl.BlockSpec((HID, OUT_CH), lambda i: (0, 0)),
            pl.BlockSpec((1, OUT_CH), lambda i: (0, 0)),
        ],
        out_specs=pl.BlockSpec((_BLK, OUT_CH), lambda i: (i, 0)),
        out_shape=jax.ShapeDtypeStruct((N_NODES, OUT_CH), jnp.float32),
    )(aggp, g2, dinv, w2, b2)


def kernel(x, edge_index, W1, b1, W2, b2):
    src = edge_index[0].astype(jnp.int32)
    dst = edge_index[1].astype(jnp.int32)
    pad = _E_PAD - N_EDGES
    src_r = jnp.concatenate([src, jnp.zeros((pad,), jnp.int32)]).reshape(_ROWS, _CHUNK)
    trash = _TRASH + jnp.arange(pad, dtype=jnp.int32) % (_NPAD - _TRASH)
    dst_r = jnp.concatenate([dst, trash]).reshape(_ROWS, _CHUNK)
    degp = _deg_kernel(dst_r)
    g1, dinv = _tc1(x, W1, degp.T)
    agg1 = _agg_kernel(src_r, dst_r, g1)
    g2 = _tc2(agg1, g1, dinv, b1.reshape(1, HID))
    agg2 = _agg_kernel(src_r, dst_r, g2)
    return _tc3(agg2, g2, dinv, W2, b2.reshape(1, OUT_CH))


# speedup vs baseline: 1.0547x; 1.0251x over previous
import functools

import jax
import jax.numpy as jnp
from jax import lax
from jax.experimental import pallas as pl
from jax.experimental.pallas import tpu as pltpu
from jax.experimental.pallas import tpu_sc as plsc

N_NODES = 10000
N_EDGES = 320000
IN_CH, HID, OUT_CH = 128, 16, 128

_NC, _NS = 2, 16
_NW = _NC * _NS
_CHUNK = 128
_CPT = -(-(-(-N_EDGES // (_NW * _CHUNK))) // 8) * 8
_E_PAD = _NW * _CPT * _CHUNK
_ROWS = _E_PAD // _CHUNK
_TRASH = N_NODES
_NPAD = 10240
_D = 8
_C0 = 112
_C1 = 2 * _CPT - _C0
_CMAX = max(_C0, _C1)
_RPT = _NPAD // _NS

_mesh = plsc.VectorSubcoreMesh(
    core_axis_name="c", subcore_axis_name="s", num_cores=_NC, num_subcores=_NS
)


@functools.partial(
    pl.kernel,
    out_type=jax.ShapeDtypeStruct((_NC, _NPAD), jnp.float32),
    mesh=_mesh,
    scratch_types=[
        pltpu.VMEM((_CPT, _CHUNK), jnp.int32),
        pltpu.VMEM((_CHUNK,), jnp.float32),
        pltpu.VMEM((_RPT,), jnp.float32),
        pltpu.VMEM_SHARED((_NPAD,), jnp.float32),
    ],
)
def _deg_kernel(dst_hbm, out_hbm, didx, ones, zbuf, deg_sh):
    cid = lax.axis_index("c")
    sid = lax.axis_index("s")
    tid = cid * _NS + sid

    def _init(i, _):
        ones[pl.ds(i * 16, 16)] = jnp.ones((16,), jnp.float32)
        zbuf[pl.ds(i * 16, 16)] = jnp.zeros((16,), jnp.float32)
        return 0

    lax.fori_loop(0, _CHUNK // 16, _init, 0)

    def _zinit(i, _):
        zbuf[pl.ds(i * 16, 16)] = jnp.zeros((16,), jnp.float32)
        return 0

    lax.fori_loop(0, _RPT // 16, _zinit, 0)

    pltpu.sync_copy(zbuf, deg_sh.at[pl.ds(sid * _RPT, _RPT)])
    pltpu.sync_copy(dst_hbm.at[pl.ds(tid * _CPT, _CPT)], didx)
    plsc.subcore_barrier()

    def _scat(j, _):
        pltpu.sync_copy(ones, deg_sh.at[didx.at[j]], add=True)
        return 0

    lax.fori_loop(0, _CPT, _scat, 0)
    plsc.subcore_barrier()

    pltpu.sync_copy(deg_sh.at[pl.ds(sid * _RPT, _RPT)], zbuf)
    pltpu.sync_copy(zbuf, out_hbm.at[cid, pl.ds(sid * _RPT, _RPT)])


@functools.partial(
    pl.kernel,
    out_type=jax.ShapeDtypeStruct((_NC, _NPAD, HID), jnp.float32),
    mesh=_mesh,
    compiler_params=pltpu.CompilerParams(use_tc_tiling_on_sc=False),
    scratch_types=[
        pltpu.VMEM((_CMAX, _CHUNK), jnp.int32),
        pltpu.VMEM((_CMAX, _CHUNK), jnp.int32),
        pltpu.VMEM((_D, _CHUNK, HID), jnp.float32),
        pltpu.VMEM((_RPT, HID), jnp.float32),
        pltpu.VMEM_SHARED((_NPAD, HID), jnp.float32),
        pltpu.SemaphoreType.DMA((_D,)),
        pltpu.SemaphoreType.DMA((_D,)),
    ],
)
def _agg_kernel(src_hbm, dst_hbm, g_hbm, out_hbm, sidx, didx, rows, zbuf, agg_sh,
                gsem, ssem):
    cid = lax.axis_index("c")
    sid = lax.axis_index("s")
    my_cnt = jnp.where(cid == 0, _C0, _C1)
    base_row = jnp.where(cid == 0, sid * _C0, _NS * _C0 + sid * _C1)

    def _zinit(i, _):
        zbuf[i, :] = jnp.zeros((HID,), jnp.float32)
        return 0

    lax.fori_loop(0, _RPT, _zinit, 0)

    pltpu.sync_copy(zbuf, agg_sh.at[pl.ds(sid * _RPT, _RPT)])

    @pl.when(cid == 0)
    def _():
        pltpu.sync_copy(src_hbm.at[pl.ds(base_row, _C0)], sidx.at[pl.ds(0, _C0)])
        pltpu.sync_copy(dst_hbm.at[pl.ds(base_row, _C0)], didx.at[pl.ds(0, _C0)])

    @pl.when(cid == 1)
    def _():
        pltpu.sync_copy(src_hbm.at[pl.ds(base_row, _C1)], sidx.at[pl.ds(0, _C1)])
        pltpu.sync_copy(dst_hbm.at[pl.ds(base_row, _C1)], didx.at[pl.ds(0, _C1)])

    plsc.subcore_barrier()

    def _group(g, _):
        base = g * _D
        for b in range(_D):
            j = base + b

            @pl.when(g > 0)
            def _():
                pltpu.make_async_copy(
                    rows.at[b], agg_sh.at[didx.at[j]], ssem.at[b]
                ).wait()

            pltpu.async_copy(g_hbm.at[sidx.at[j]], rows.at[b], gsem.at[b])
        for b in range(_D):
            j = base + b
            pltpu.make_async_copy(g_hbm.at[sidx.at[j]], rows.at[b], gsem.at[b]).wait()
            pltpu.make_async_copy(rows.at[b], agg_sh.at[didx.at[j]], ssem.at[b]).start(
                add=True
            )
        return 0

    lax.fori_loop(0, my_cnt // _D, _group, 0)
    for b in range(_D):
        pltpu.make_async_copy(rows.at[b], agg_sh.at[didx.at[b]], ssem.at[b]).wait()
    plsc.subcore_barrier()

    pltpu.sync_copy(agg_sh.at[pl.ds(sid * _RPT, _RPT)], zbuf)
    pltpu.sync_copy(zbuf, out_hbm.at[cid, pl.ds(sid * _RPT, _RPT)])


_BLK = 1000
_GRID = N_NODES // _BLK


def _tc1_body(x_ref, w1_ref, degp_ref, g1_ref, dinv_ref):
    deg = 1.0 + degp_ref[:, 0] + degp_ref[:, 1]
    dinv = lax.rsqrt(deg)[:, None]
    h = jnp.dot(x_ref[...], w1_ref[...], preferred_element_type=jnp.float32)
    g1_ref[...] = h * dinv
    dinv_ref[...] = dinv


def _tc1(x, w1, degp):
    return pl.pallas_call(
        _tc1_body,
        grid=(_GRID,),
        in_specs=[
            pl.BlockSpec((_BLK, IN_CH), lambda i: (i, 0)),
            pl.BlockSpec((IN_CH, HID), lambda i: (0, 0)),
            pl.BlockSpec((_BLK, 2), lambda i: (i, 0)),
        ],
        out_specs=[
            pl.BlockSpec((_BLK, HID), lambda i: (i, 0)),
            pl.BlockSpec((_BLK, 1), lambda i: (i, 0)),
        ],
        out_shape=[
            jax.ShapeDtypeStruct((N_NODES, HID), jnp.float32),
            jax.ShapeDtypeStruct((N_NODES, 1), jnp.float32),
        ],
    )(x, w1, degp)


def _tc2_body(aggp_ref, g1_ref, dinv_ref, b1_ref, g2_ref):
    agg = aggp_ref[0] + aggp_ref[1] + g1_ref[...]
    dinv = dinv_ref[...]
    h1 = jnp.maximum(dinv * agg + b1_ref[...], 0.0)
    g2_ref[...] = dinv * h1


def _tc2(aggp, g1, dinv, b1):
    return pl.pallas_call(
        _tc2_body,
        grid=(_GRID,),
        in_specs=[
            pl.BlockSpec((2, _BLK, HID), lambda i: (0, i, 0)),
            pl.BlockSpec((_BLK, HID), lambda i: (i, 0)),
            pl.BlockSpec((_BLK, 1), lambda i: (i, 0)),
            pl.BlockSpec((1, HID), lambda i: (0, 0)),
        ],
        out_specs=pl.BlockSpec((_BLK, HID), lambda i: (i, 0)),
        out_shape=jax.ShapeDtypeStruct((N_NODES, HID), jnp.float32),
    )(aggp, g1, dinv, b1)


def _tc3_body(aggp_ref, g2_ref, dinv_ref, w2_ref, b2_ref, out_ref):
    p = dinv_ref[...] * (aggp_ref[0] + aggp_ref[1] + g2_ref[...])
    out_ref[...] = (
        jnp.dot(p, w2_ref[...], preferred_element_type=jnp.float32) + b2_ref[...]
    )


def _tc3(aggp, g2, dinv, w2, b2):
    return pl.pallas_call(
        _tc3_body,
        grid=(_GRID,),
        in_specs=[
            pl.BlockSpec((2, _BLK, HID), lambda i: (0, i, 0)),
            pl.BlockSpec((_BLK, HID), lambda i: (i, 0)),
            pl.BlockSpec((_BLK, 1), lambda i: (i, 0)),
            pl.BlockSpec((HID, OUT_CH), lambda i: (0, 0)),
            pl.BlockSpec((1, OUT_CH), lambda i: (0, 0)),
        ],
        out_specs=pl.BlockSpec((_BLK, OUT_CH), lambda i: (i, 0)),
        out_shape=jax.ShapeDtypeStruct((N_NODES, OUT_CH), jnp.float32),
    )(aggp, g2, dinv, w2, b2)


def kernel(x, edge_index, W1, b1, W2, b2):
    src = edge_index[0].astype(jnp.int32)
    dst = edge_index[1].astype(jnp.int32)
    pad = _E_PAD - N_EDGES
    src_r = jnp.concatenate([src, jnp.zeros((pad,), jnp.int32)]).reshape(_ROWS, _CHUNK)
    trash = _TRASH + jnp.arange(pad, dtype=jnp.int32) % (_NPAD - _TRASH)
    dst_r = jnp.concatenate([dst, trash]).reshape(_ROWS, _CHUNK)
    degp = _deg_kernel(dst_r)
    g1, dinv = _tc1(x, W1, degp.T)
    agg1 = _agg_kernel(src_r, dst_r, g1)
    g2 = _tc2(agg1, g1, dinv, b1.reshape(1, HID))
    agg2 = _agg_kernel(src_r, dst_r, g2)
    return _tc3(agg2, g2, dinv, W2, b2.reshape(1, OUT_CH))


# speedup vs baseline: 1.0636x; 1.0085x over previous
import functools

import jax
import jax.numpy as jnp
from jax import lax
from jax.experimental import pallas as pl
from jax.experimental.pallas import tpu as pltpu
from jax.experimental.pallas import tpu_sc as plsc

N_NODES = 10000
N_EDGES = 320000
IN_CH, HID, OUT_CH = 128, 16, 128

_NC, _NS = 2, 16
_NW = _NC * _NS
_CHUNK = 128
_CPT = -(-(-(-N_EDGES // (_NW * _CHUNK))) // 8) * 8
_E_PAD = _NW * _CPT * _CHUNK
_ROWS = _E_PAD // _CHUNK
_TRASH = N_NODES
_NPAD = 10240
_D = 8
_C0 = 120
_C1 = 2 * _CPT - _C0
_CMAX = max(_C0, _C1)
_RPT = _NPAD // _NS

_mesh = plsc.VectorSubcoreMesh(
    core_axis_name="c", subcore_axis_name="s", num_cores=_NC, num_subcores=_NS
)


@functools.partial(
    pl.kernel,
    out_type=jax.ShapeDtypeStruct((_NC, _NPAD), jnp.float32),
    mesh=_mesh,
    scratch_types=[
        pltpu.VMEM((_CPT, _CHUNK), jnp.int32),
        pltpu.VMEM((_CHUNK,), jnp.float32),
        pltpu.VMEM((_RPT,), jnp.float32),
        pltpu.VMEM_SHARED((_NPAD,), jnp.float32),
    ],
)
def _deg_kernel(dst_hbm, out_hbm, didx, ones, zbuf, deg_sh):
    cid = lax.axis_index("c")
    sid = lax.axis_index("s")
    tid = cid * _NS + sid

    def _init(i, _):
        ones[pl.ds(i * 16, 16)] = jnp.ones((16,), jnp.float32)
        zbuf[pl.ds(i * 16, 16)] = jnp.zeros((16,), jnp.float32)
        return 0

    lax.fori_loop(0, _CHUNK // 16, _init, 0)

    def _zinit(i, _):
        zbuf[pl.ds(i * 16, 16)] = jnp.zeros((16,), jnp.float32)
        return 0

    lax.fori_loop(0, _RPT // 16, _zinit, 0)

    pltpu.sync_copy(zbuf, deg_sh.at[pl.ds(sid * _RPT, _RPT)])
    pltpu.sync_copy(dst_hbm.at[pl.ds(tid * _CPT, _CPT)], didx)
    plsc.subcore_barrier()

    def _scat(j, _):
        pltpu.sync_copy(ones, deg_sh.at[didx.at[j]], add=True)
        return 0

    lax.fori_loop(0, _CPT, _scat, 0)
    plsc.subcore_barrier()

    pltpu.sync_copy(deg_sh.at[pl.ds(sid * _RPT, _RPT)], zbuf)
    pltpu.sync_copy(zbuf, out_hbm.at[cid, pl.ds(sid * _RPT, _RPT)])


@functools.partial(
    pl.kernel,
    out_type=jax.ShapeDtypeStruct((_NC, _NPAD, HID), jnp.float32),
    mesh=_mesh,
    compiler_params=pltpu.CompilerParams(use_tc_tiling_on_sc=False),
    scratch_types=[
        pltpu.VMEM((_CMAX, _CHUNK), jnp.int32),
        pltpu.VMEM((_CMAX, _CHUNK), jnp.int32),
        pltpu.VMEM((_D, _CHUNK, HID), jnp.float32),
        pltpu.VMEM((_RPT, HID), jnp.float32),
        pltpu.VMEM_SHARED((_NPAD, HID), jnp.float32),
        pltpu.SemaphoreType.DMA((_D,)),
        pltpu.SemaphoreType.DMA((_D,)),
    ],
)
def _agg_kernel(src_hbm, dst_hbm, g_hbm, out_hbm, sidx, didx, rows, zbuf, agg_sh,
                gsem, ssem):
    cid = lax.axis_index("c")
    sid = lax.axis_index("s")
    my_cnt = jnp.where(cid == 0, _C0, _C1)
    base_row = jnp.where(cid == 0, sid * _C0, _NS * _C0 + sid * _C1)

    def _zinit(i, _):
        zbuf[i, :] = jnp.zeros((HID,), jnp.float32)
        return 0

    lax.fori_loop(0, _RPT, _zinit, 0)

    pltpu.sync_copy(zbuf, agg_sh.at[pl.ds(sid * _RPT, _RPT)])

    @pl.when(cid == 0)
    def _():
        pltpu.sync_copy(src_hbm.at[pl.ds(base_row, _C0)], sidx.at[pl.ds(0, _C0)])
        pltpu.sync_copy(dst_hbm.at[pl.ds(base_row, _C0)], didx.at[pl.ds(0, _C0)])

    @pl.when(cid == 1)
    def _():
        pltpu.sync_copy(src_hbm.at[pl.ds(base_row, _C1)], sidx.at[pl.ds(0, _C1)])
        pltpu.sync_copy(dst_hbm.at[pl.ds(base_row, _C1)], didx.at[pl.ds(0, _C1)])

    plsc.subcore_barrier()

    def _group(g, _):
        base = g * _D
        for b in range(_D):
            j = base + b

            @pl.when(g > 0)
            def _():
                pltpu.make_async_copy(
                    rows.at[b], agg_sh.at[didx.at[j]], ssem.at[b]
                ).wait()

            pltpu.async_copy(g_hbm.at[sidx.at[j]], rows.at[b], gsem.at[b])
        for b in range(_D):
            j = base + b
            pltpu.make_async_copy(g_hbm.at[sidx.at[j]], rows.at[b], gsem.at[b]).wait()
            pltpu.make_async_copy(rows.at[b], agg_sh.at[didx.at[j]], ssem.at[b]).start(
                add=True
            )
        return 0

    lax.fori_loop(0, my_cnt // _D, _group, 0)
    for b in range(_D):
        pltpu.make_async_copy(rows.at[b], agg_sh.at[didx.at[b]], ssem.at[b]).wait()
    plsc.subcore_barrier()

    pltpu.sync_copy(agg_sh.at[pl.ds(sid * _RPT, _RPT)], zbuf)
    pltpu.sync_copy(zbuf, out_hbm.at[cid, pl.ds(sid * _RPT, _RPT)])


_BLK = 1000
_GRID = N_NODES // _BLK


def _tc1_body(x_ref, w1_ref, degp_ref, g1_ref, dinv_ref):
    deg = 1.0 + degp_ref[:, 0] + degp_ref[:, 1]
    dinv = lax.rsqrt(deg)[:, None]
    h = jnp.dot(x_ref[...], w1_ref[...], preferred_element_type=jnp.float32)
    g1_ref[...] = h * dinv
    dinv_ref[...] = dinv


def _tc1(x, w1, degp):
    return pl.pallas_call(
        _tc1_body,
        grid=(_GRID,),
        in_specs=[
            pl.BlockSpec((_BLK, IN_CH), lambda i: (i, 0)),
            pl.BlockSpec((IN_CH, HID), lambda i: (0, 0)),
            pl.BlockSpec((_BLK, 2), lambda i: (i, 0)),
        ],
        out_specs=[
            pl.BlockSpec((_BLK, HID), lambda i: (i, 0)),
            pl.BlockSpec((_BLK, 1), lambda i: (i, 0)),
        ],
        out_shape=[
            jax.ShapeDtypeStruct((N_NODES, HID), jnp.float32),
            jax.ShapeDtypeStruct((N_NODES, 1), jnp.float32),
        ],
    )(x, w1, degp)


def _tc2_body(aggp_ref, g1_ref, dinv_ref, b1_ref, g2_ref):
    agg = aggp_ref[0] + aggp_ref[1] + g1_ref[...]
    dinv = dinv_ref[...]
    h1 = jnp.maximum(dinv * agg + b1_ref[...], 0.0)
    g2_ref[...] = dinv * h1


def _tc2(aggp, g1, dinv, b1):
    return pl.pallas_call(
        _tc2_body,
        grid=(_GRID,),
        in_specs=[
            pl.BlockSpec((2, _BLK, HID), lambda i: (0, i, 0)),
            pl.BlockSpec((_BLK, HID), lambda i: (i, 0)),
            pl.BlockSpec((_BLK, 1), lambda i: (i, 0)),
            pl.BlockSpec((1, HID), lambda i: (0, 0)),
        ],
        out_specs=pl.BlockSpec((_BLK, HID), lambda i: (i, 0)),
        out_shape=jax.ShapeDtypeStruct((N_NODES, HID), jnp.float32),
    )(aggp, g1, dinv, b1)


def _tc3_body(aggp_ref, g2_ref, dinv_ref, w2_ref, b2_ref, out_ref):
    p = dinv_ref[...] * (aggp_ref[0] + aggp_ref[1] + g2_ref[...])
    out_ref[...] = (
        jnp.dot(p, w2_ref[...], preferred_element_type=jnp.float32) + b2_ref[...]
    )


def _tc3(aggp, g2, dinv, w2, b2):
    return pl.pallas_call(
        _tc3_body,
        grid=(_GRID,),
        in_specs=[
            pl.BlockSpec((2, _BLK, HID), lambda i: (0, i, 0)),
            pl.BlockSpec((_BLK, HID), lambda i: (i, 0)),
            pl.BlockSpec((_BLK, 1), lambda i: (i, 0)),
            pl.BlockSpec((HID, OUT_CH), lambda i: (0, 0)),
            pl.BlockSpec((1, OUT_CH), lambda i: (0, 0)),
        ],
        out_specs=pl.BlockSpec((_BLK, OUT_CH), lambda i: (i, 0)),
        out_shape=jax.ShapeDtypeStruct((N_NODES, OUT_CH), jnp.float32),
    )(aggp, g2, dinv, w2, b2)


def kernel(x, edge_index, W1, b1, W2, b2):
    src = edge_index[0].astype(jnp.int32)
    dst = edge_index[1].astype(jnp.int32)
    pad = _E_PAD - N_EDGES
    src_r = jnp.concatenate([src, jnp.zeros((pad,), jnp.int32)]).reshape(_ROWS, _CHUNK)
    trash = _TRASH + jnp.arange(pad, dtype=jnp.int32) % (_NPAD - _TRASH)
    dst_r = jnp.concatenate([dst, trash]).reshape(_ROWS, _CHUNK)
    degp = _deg_kernel(dst_r)
    g1, dinv = _tc1(x, W1, degp.T)
    agg1 = _agg_kernel(src_r, dst_r, g1)
    g2 = _tc2(agg1, g1, dinv, b1.reshape(1, HID))
    agg2 = _agg_kernel(src_r, dst_r, g2)
    return _tc3(agg2, g2, dinv, W2, b2.reshape(1, OUT_CH))


# speedup vs baseline: 1.0763x; 1.0119x over previous
import functools

import jax
import jax.numpy as jnp
from jax import lax
from jax.experimental import pallas as pl
from jax.experimental.pallas import tpu as pltpu
from jax.experimental.pallas import tpu_sc as plsc

N_NODES = 10000
N_EDGES = 320000
IN_CH, HID, OUT_CH = 128, 16, 128

_NC, _NS = 2, 16
_NW = _NC * _NS
_CHUNK = 128
_CPT = -(-(-(-N_EDGES // (_NW * _CHUNK))) // 8) * 8
_E_PAD = _NW * _CPT * _CHUNK
_ROWS = _E_PAD // _CHUNK
_TRASH = N_NODES
_NPAD = 10240
_D = 8
_C0 = 128
_C1 = 2 * _CPT - _C0
_CMAX = max(_C0, _C1)
_RPT = _NPAD // _NS

_mesh = plsc.VectorSubcoreMesh(
    core_axis_name="c", subcore_axis_name="s", num_cores=_NC, num_subcores=_NS
)


@functools.partial(
    pl.kernel,
    out_type=jax.ShapeDtypeStruct((_NC, _NPAD), jnp.float32),
    mesh=_mesh,
    scratch_types=[
        pltpu.VMEM((_CPT, _CHUNK), jnp.int32),
        pltpu.VMEM((_CHUNK,), jnp.float32),
        pltpu.VMEM((_RPT,), jnp.float32),
        pltpu.VMEM_SHARED((_NPAD,), jnp.float32),
    ],
)
def _deg_kernel(dst_hbm, out_hbm, didx, ones, zbuf, deg_sh):
    cid = lax.axis_index("c")
    sid = lax.axis_index("s")
    tid = cid * _NS + sid

    def _init(i, _):
        ones[pl.ds(i * 16, 16)] = jnp.ones((16,), jnp.float32)
        zbuf[pl.ds(i * 16, 16)] = jnp.zeros((16,), jnp.float32)
        return 0

    lax.fori_loop(0, _CHUNK // 16, _init, 0)

    def _zinit(i, _):
        zbuf[pl.ds(i * 16, 16)] = jnp.zeros((16,), jnp.float32)
        return 0

    lax.fori_loop(0, _RPT // 16, _zinit, 0)

    pltpu.sync_copy(zbuf, deg_sh.at[pl.ds(sid * _RPT, _RPT)])
    pltpu.sync_copy(dst_hbm.at[pl.ds(tid * _CPT, _CPT)], didx)
    plsc.subcore_barrier()

    def _scat(j, _):
        pltpu.sync_copy(ones, deg_sh.at[didx.at[j]], add=True)
        return 0

    lax.fori_loop(0, _CPT, _scat, 0)
    plsc.subcore_barrier()

    pltpu.sync_copy(deg_sh.at[pl.ds(sid * _RPT, _RPT)], zbuf)
    pltpu.sync_copy(zbuf, out_hbm.at[cid, pl.ds(sid * _RPT, _RPT)])


@functools.partial(
    pl.kernel,
    out_type=jax.ShapeDtypeStruct((_NC, _NPAD, HID), jnp.float32),
    mesh=_mesh,
    compiler_params=pltpu.CompilerParams(use_tc_tiling_on_sc=False),
    scratch_types=[
        pltpu.VMEM((_CMAX, _CHUNK), jnp.int32),
        pltpu.VMEM((_CMAX, _CHUNK), jnp.int32),
        pltpu.VMEM((_D, _CHUNK, HID), jnp.float32),
        pltpu.VMEM((_RPT, HID), jnp.float32),
        pltpu.VMEM_SHARED((_NPAD, HID), jnp.float32),
        pltpu.SemaphoreType.DMA((_D,)),
        pltpu.SemaphoreType.DMA((_D,)),
    ],
)
def _agg_kernel(src_hbm, dst_hbm, g_hbm, out_hbm, sidx, didx, rows, zbuf, agg_sh,
                gsem, ssem):
    cid = lax.axis_index("c")
    sid = lax.axis_index("s")
    my_cnt = jnp.where(cid == 0, _C0, _C1)
    base_row = jnp.where(cid == 0, sid * _C0, _NS * _C0 + sid * _C1)

    def _zinit(i, _):
        zbuf[i, :] = jnp.zeros((HID,), jnp.float32)
        return 0

    lax.fori_loop(0, _RPT, _zinit, 0)

    pltpu.sync_copy(zbuf, agg_sh.at[pl.ds(sid * _RPT, _RPT)])

    @pl.when(cid == 0)
    def _():
        pltpu.sync_copy(src_hbm.at[pl.ds(base_row, _C0)], sidx.at[pl.ds(0, _C0)])
        pltpu.sync_copy(dst_hbm.at[pl.ds(base_row, _C0)], didx.at[pl.ds(0, _C0)])

    @pl.when(cid == 1)
    def _():
        pltpu.sync_copy(src_hbm.at[pl.ds(base_row, _C1)], sidx.at[pl.ds(0, _C1)])
        pltpu.sync_copy(dst_hbm.at[pl.ds(base_row, _C1)], didx.at[pl.ds(0, _C1)])

    plsc.subcore_barrier()

    def _group(g, _):
        base = g * _D
        for b in range(_D):
            j = base + b

            @pl.when(g > 0)
            def _():
                pltpu.make_async_copy(
                    rows.at[b], agg_sh.at[didx.at[j]], ssem.at[b]
                ).wait()

            pltpu.async_copy(g_hbm.at[sidx.at[j]], rows.at[b], gsem.at[b])
        for b in range(_D):
            j = base + b
            pltpu.make_async_copy(g_hbm.at[sidx.at[j]], rows.at[b], gsem.at[b]).wait()
            pltpu.make_async_copy(rows.at[b], agg_sh.at[didx.at[j]], ssem.at[b]).start(
                add=True
            )
        return 0

    lax.fori_loop(0, my_cnt // _D, _group, 0)
    for b in range(_D):
        pltpu.make_async_copy(rows.at[b], agg_sh.at[didx.at[b]], ssem.at[b]).wait()
    plsc.subcore_barrier()

    pltpu.sync_copy(agg_sh.at[pl.ds(sid * _RPT, _RPT)], zbuf)
    pltpu.sync_copy(zbuf, out_hbm.at[cid, pl.ds(sid * _RPT, _RPT)])


_BLK = 1000
_GRID = N_NODES // _BLK


def _tc1_body(x_ref, w1_ref, degp_ref, g1_ref, dinv_ref):
    deg = 1.0 + degp_ref[:, 0] + degp_ref[:, 1]
    dinv = lax.rsqrt(deg)[:, None]
    h = jnp.dot(x_ref[...], w1_ref[...], preferred_element_type=jnp.float32)
    g1_ref[...] = h * dinv
    dinv_ref[...] = dinv


def _tc1(x, w1, degp):
    return pl.pallas_call(
        _tc1_body,
        grid=(_GRID,),
        in_specs=[
            pl.BlockSpec((_BLK, IN_CH), lambda i: (i, 0)),
            pl.BlockSpec((IN_CH, HID), lambda i: (0, 0)),
            pl.BlockSpec((_BLK, 2), lambda i: (i, 0)),
        ],
        out_specs=[
            pl.BlockSpec((_BLK, HID), lambda i: (i, 0)),
            pl.BlockSpec((_BLK, 1), lambda i: (i, 0)),
        ],
        out_shape=[
            jax.ShapeDtypeStruct((N_NODES, HID), jnp.float32),
            jax.ShapeDtypeStruct((N_NODES, 1), jnp.float32),
        ],
    )(x, w1, degp)


def _tc2_body(aggp_ref, g1_ref, dinv_ref, b1_ref, g2_ref):
    agg = aggp_ref[0] + aggp_ref[1] + g1_ref[...]
    dinv = dinv_ref[...]
    h1 = jnp.maximum(dinv * agg + b1_ref[...], 0.0)
    g2_ref[...] = dinv * h1


def _tc2(aggp, g1, dinv, b1):
    return pl.pallas_call(
        _tc2_body,
        grid=(_GRID,),
        in_specs=[
            pl.BlockSpec((2, _BLK, HID), lambda i: (0, i, 0)),
            pl.BlockSpec((_BLK, HID), lambda i: (i, 0)),
            pl.BlockSpec((_BLK, 1), lambda i: (i, 0)),
            pl.BlockSpec((1, HID), lambda i: (0, 0)),
        ],
        out_specs=pl.BlockSpec((_BLK, HID), lambda i: (i, 0)),
        out_shape=jax.ShapeDtypeStruct((N_NODES, HID), jnp.float32),
    )(aggp, g1, dinv, b1)


def _tc3_body(aggp_ref, g2_ref, dinv_ref, w2_ref, b2_ref, out_ref):
    p = dinv_ref[...] * (aggp_ref[0] + aggp_ref[1] + g2_ref[...])
    out_ref[...] = (
        jnp.dot(p, w2_ref[...], preferred_element_type=jnp.float32) + b2_ref[...]
    )


def _tc3(aggp, g2, dinv, w2, b2):
    return pl.pallas_call(
        _tc3_body,
        grid=(_GRID,),
        in_specs=[
            pl.BlockSpec((2, _BLK, HID), lambda i: (0, i, 0)),
            pl.BlockSpec((_BLK, HID), lambda i: (i, 0)),
            pl.BlockSpec((_BLK, 1), lambda i: (i, 0)),
            pl.BlockSpec((HID, OUT_CH), lambda i: (0, 0)),
            pl.BlockSpec((1, OUT_CH), lambda i: (0, 0)),
        ],
        out_specs=pl.BlockSpec((_BLK, OUT_CH), lambda i: (i, 0)),
        out_shape=jax.ShapeDtypeStruct((N_NODES, OUT_CH), jnp.float32),
    )(aggp, g2, dinv, w2, b2)


def kernel(x, edge_index, W1, b1, W2, b2):
    src = edge_index[0].astype(jnp.int32)
    dst = edge_index[1].astype(jnp.int32)
    pad = _E_PAD - N_EDGES
    src_r = jnp.concatenate([src, jnp.zeros((pad,), jnp.int32)]).reshape(_ROWS, _CHUNK)
    trash = _TRASH + jnp.arange(pad, dtype=jnp.int32) % (_NPAD - _TRASH)
    dst_r = jnp.concatenate([dst, trash]).reshape(_ROWS, _CHUNK)
    degp = _deg_kernel(dst_r)
    g1, dinv = _tc1(x, W1, degp.T)
    agg1 = _agg_kernel(src_r, dst_r, g1)
    g2 = _tc2(agg1, g1, dinv, b1.reshape(1, HID))
    agg2 = _agg_kernel(src_r, dst_r, g2)
    return _tc3(agg2, g2, dinv, W2, b2.reshape(1, OUT_CH))


# speedup vs baseline: 1.1065x; 1.0280x over previous
import functools

import jax
import jax.numpy as jnp
from jax import lax
from jax.experimental import pallas as pl
from jax.experimental.pallas import tpu as pltpu
from jax.experimental.pallas import tpu_sc as plsc

N_NODES = 10000
N_EDGES = 320000
IN_CH, HID, OUT_CH = 128, 16, 128

_NC, _NS = 2, 16
_NW = _NC * _NS
_CHUNK = 128
_CPT = -(-(-(-N_EDGES // (_NW * _CHUNK))) // 8) * 8
_E_PAD = _NW * _CPT * _CHUNK
_ROWS = _E_PAD // _CHUNK
_TRASH = N_NODES
_NPAD = 10240
_D = 8
_C0 = 144
_C1 = 2 * _CPT - _C0
_CMAX = max(_C0, _C1)
_RPT = _NPAD // _NS

_mesh = plsc.VectorSubcoreMesh(
    core_axis_name="c", subcore_axis_name="s", num_cores=_NC, num_subcores=_NS
)


@functools.partial(
    pl.kernel,
    out_type=jax.ShapeDtypeStruct((_NC, _NPAD), jnp.float32),
    mesh=_mesh,
    scratch_types=[
        pltpu.VMEM((_CPT, _CHUNK), jnp.int32),
        pltpu.VMEM((_CHUNK,), jnp.float32),
        pltpu.VMEM((_RPT,), jnp.float32),
        pltpu.VMEM_SHARED((_NPAD,), jnp.float32),
    ],
)
def _deg_kernel(dst_hbm, out_hbm, didx, ones, zbuf, deg_sh):
    cid = lax.axis_index("c")
    sid = lax.axis_index("s")
    tid = cid * _NS + sid

    def _init(i, _):
        ones[pl.ds(i * 16, 16)] = jnp.ones((16,), jnp.float32)
        zbuf[pl.ds(i * 16, 16)] = jnp.zeros((16,), jnp.float32)
        return 0

    lax.fori_loop(0, _CHUNK // 16, _init, 0)

    def _zinit(i, _):
        zbuf[pl.ds(i * 16, 16)] = jnp.zeros((16,), jnp.float32)
        return 0

    lax.fori_loop(0, _RPT // 16, _zinit, 0)

    pltpu.sync_copy(zbuf, deg_sh.at[pl.ds(sid * _RPT, _RPT)])
    pltpu.sync_copy(dst_hbm.at[pl.ds(tid * _CPT, _CPT)], didx)
    plsc.subcore_barrier()

    def _scat(j, _):
        pltpu.sync_copy(ones, deg_sh.at[didx.at[j]], add=True)
        return 0

    lax.fori_loop(0, _CPT, _scat, 0)
    plsc.subcore_barrier()

    pltpu.sync_copy(deg_sh.at[pl.ds(sid * _RPT, _RPT)], zbuf)
    pltpu.sync_copy(zbuf, out_hbm.at[cid, pl.ds(sid * _RPT, _RPT)])


@functools.partial(
    pl.kernel,
    out_type=jax.ShapeDtypeStruct((_NC, _NPAD, HID), jnp.float32),
    mesh=_mesh,
    compiler_params=pltpu.CompilerParams(use_tc_tiling_on_sc=False),
    scratch_types=[
        pltpu.VMEM((_CMAX, _CHUNK), jnp.int32),
        pltpu.VMEM((_CMAX, _CHUNK), jnp.int32),
        pltpu.VMEM((_D, _CHUNK, HID), jnp.float32),
        pltpu.VMEM((_RPT, HID), jnp.float32),
        pltpu.VMEM_SHARED((_NPAD, HID), jnp.float32),
        pltpu.SemaphoreType.DMA((_D,)),
        pltpu.SemaphoreType.DMA((_D,)),
    ],
)
def _agg_kernel(src_hbm, dst_hbm, g_hbm, out_hbm, sidx, didx, rows, zbuf, agg_sh,
                gsem, ssem):
    cid = lax.axis_index("c")
    sid = lax.axis_index("s")
    my_cnt = jnp.where(cid == 0, _C0, _C1)
    base_row = jnp.where(cid == 0, sid * _C0, _NS * _C0 + sid * _C1)

    def _zinit(i, _):
        zbuf[i, :] = jnp.zeros((HID,), jnp.float32)
        return 0

    lax.fori_loop(0, _RPT, _zinit, 0)

    pltpu.sync_copy(zbuf, agg_sh.at[pl.ds(sid * _RPT, _RPT)])

    @pl.when(cid == 0)
    def _():
        pltpu.sync_copy(src_hbm.at[pl.ds(base_row, _C0)], sidx.at[pl.ds(0, _C0)])
        pltpu.sync_copy(dst_hbm.at[pl.ds(base_row, _C0)], didx.at[pl.ds(0, _C0)])

    @pl.when(cid == 1)
    def _():
        pltpu.sync_copy(src_hbm.at[pl.ds(base_row, _C1)], sidx.at[pl.ds(0, _C1)])
        pltpu.sync_copy(dst_hbm.at[pl.ds(base_row, _C1)], didx.at[pl.ds(0, _C1)])

    plsc.subcore_barrier()

    def _group(g, _):
        base = g * _D
        for b in range(_D):
            j = base + b

            @pl.when(g > 0)
            def _():
                pltpu.make_async_copy(
                    rows.at[b], agg_sh.at[didx.at[j]], ssem.at[b]
                ).wait()

            pltpu.async_copy(g_hbm.at[sidx.at[j]], rows.at[b], gsem.at[b])
        for b in range(_D):
            j = base + b
            pltpu.make_async_copy(g_hbm.at[sidx.at[j]], rows.at[b], gsem.at[b]).wait()
            pltpu.make_async_copy(rows.at[b], agg_sh.at[didx.at[j]], ssem.at[b]).start(
                add=True
            )
        return 0

    lax.fori_loop(0, my_cnt // _D, _group, 0)
    for b in range(_D):
        pltpu.make_async_copy(rows.at[b], agg_sh.at[didx.at[b]], ssem.at[b]).wait()
    plsc.subcore_barrier()

    pltpu.sync_copy(agg_sh.at[pl.ds(sid * _RPT, _RPT)], zbuf)
    pltpu.sync_copy(zbuf, out_hbm.at[cid, pl.ds(sid * _RPT, _RPT)])


_BLK = 1000
_GRID = N_NODES // _BLK


def _tc1_body(x_ref, w1_ref, degp_ref, g1_ref, dinv_ref):
    deg = 1.0 + degp_ref[:, 0] + degp_ref[:, 1]
    dinv = lax.rsqrt(deg)[:, None]
    h = jnp.dot(x_ref[...], w1_ref[...], preferred_element_type=jnp.float32)
    g1_ref[...] = h * dinv
    dinv_ref[...] = dinv


def _tc1(x, w1, degp):
    return pl.pallas_call(
        _tc1_body,
        grid=(_GRID,),
        in_specs=[
            pl.BlockSpec((_BLK, IN_CH), lambda i: (i, 0)),
            pl.BlockSpec((IN_CH, HID), lambda i: (0, 0)),
            pl.BlockSpec((_BLK, 2), lambda i: (i, 0)),
        ],
        out_specs=[
            pl.BlockSpec((_BLK, HID), lambda i: (i, 0)),
            pl.BlockSpec((_BLK, 1), lambda i: (i, 0)),
        ],
        out_shape=[
            jax.ShapeDtypeStruct((N_NODES, HID), jnp.float32),
            jax.ShapeDtypeStruct((N_NODES, 1), jnp.float32),
        ],
    )(x, w1, degp)


def _tc2_body(aggp_ref, g1_ref, dinv_ref, b1_ref, g2_ref):
    agg = aggp_ref[0] + aggp_ref[1] + g1_ref[...]
    dinv = dinv_ref[...]
    h1 = jnp.maximum(dinv * agg + b1_ref[...], 0.0)
    g2_ref[...] = dinv * h1


def _tc2(aggp, g1, dinv, b1):
    return pl.pallas_call(
        _tc2_body,
        grid=(_GRID,),
        in_specs=[
            pl.BlockSpec((2, _BLK, HID), lambda i: (0, i, 0)),
            pl.BlockSpec((_BLK, HID), lambda i: (i, 0)),
            pl.BlockSpec((_BLK, 1), lambda i: (i, 0)),
            pl.BlockSpec((1, HID), lambda i: (0, 0)),
        ],
        out_specs=pl.BlockSpec((_BLK, HID), lambda i: (i, 0)),
        out_shape=jax.ShapeDtypeStruct((N_NODES, HID), jnp.float32),
    )(aggp, g1, dinv, b1)


def _tc3_body(aggp_ref, g2_ref, dinv_ref, w2_ref, b2_ref, out_ref):
    p = dinv_ref[...] * (aggp_ref[0] + aggp_ref[1] + g2_ref[...])
    out_ref[...] = (
        jnp.dot(p, w2_ref[...], preferred_element_type=jnp.float32) + b2_ref[...]
    )


def _tc3(aggp, g2, dinv, w2, b2):
    return pl.pallas_call(
        _tc3_body,
        grid=(_GRID,),
        in_specs=[
            pl.BlockSpec((2, _BLK, HID), lambda i: (0, i, 0)),
            pl.BlockSpec((_BLK, HID), lambda i: (i, 0)),
            pl.BlockSpec((_BLK, 1), lambda i: (i, 0)),
            pl.BlockSpec((HID, OUT_CH), lambda i: (0, 0)),
            pl.BlockSpec((1, OUT_CH), lambda i: (0, 0)),
        ],
        out_specs=pl.BlockSpec((_BLK, OUT_CH), lambda i: (i, 0)),
        out_shape=jax.ShapeDtypeStruct((N_NODES, OUT_CH), jnp.float32),
    )(aggp, g2, dinv, w2, b2)


def kernel(x, edge_index, W1, b1, W2, b2):
    src = edge_index[0].astype(jnp.int32)
    dst = edge_index[1].astype(jnp.int32)
    pad = _E_PAD - N_EDGES
    src_r = jnp.concatenate([src, jnp.zeros((pad,), jnp.int32)]).reshape(_ROWS, _CHUNK)
    trash = _TRASH + jnp.arange(pad, dtype=jnp.int32) % (_NPAD - _TRASH)
    dst_r = jnp.concatenate([dst, trash]).reshape(_ROWS, _CHUNK)
    degp = _deg_kernel(dst_r)
    g1, dinv = _tc1(x, W1, degp.T)
    agg1 = _agg_kernel(src_r, dst_r, g1)
    g2 = _tc2(agg1, g1, dinv, b1.reshape(1, HID))
    agg2 = _agg_kernel(src_r, dst_r, g2)
    return _tc3(agg2, g2, dinv, W2, b2.reshape(1, OUT_CH))


# speedup vs baseline: 1.1135x; 1.0063x over previous
import functools

import jax
import jax.numpy as jnp
from jax import lax
from jax.experimental import pallas as pl
from jax.experimental.pallas import tpu as pltpu
from jax.experimental.pallas import tpu_sc as plsc

N_NODES = 10000
N_EDGES = 320000
IN_CH, HID, OUT_CH = 128, 16, 128

_NC, _NS = 2, 16
_NW = _NC * _NS
_CHUNK = 128
_CPT = -(-(-(-N_EDGES // (_NW * _CHUNK))) // 8) * 8
_E_PAD = _NW * _CPT * _CHUNK
_ROWS = _E_PAD // _CHUNK
_TRASH = N_NODES
_NPAD = 10240
_D = 8
_C0 = 152
_C1 = 2 * _CPT - _C0
_CMAX = max(_C0, _C1)
_RPT = _NPAD // _NS

_mesh = plsc.VectorSubcoreMesh(
    core_axis_name="c", subcore_axis_name="s", num_cores=_NC, num_subcores=_NS
)


@functools.partial(
    pl.kernel,
    out_type=jax.ShapeDtypeStruct((_NC, _NPAD), jnp.float32),
    mesh=_mesh,
    scratch_types=[
        pltpu.VMEM((_CPT, _CHUNK), jnp.int32),
        pltpu.VMEM((_CHUNK,), jnp.float32),
        pltpu.VMEM((_RPT,), jnp.float32),
        pltpu.VMEM_SHARED((_NPAD,), jnp.float32),
    ],
)
def _deg_kernel(dst_hbm, out_hbm, didx, ones, zbuf, deg_sh):
    cid = lax.axis_index("c")
    sid = lax.axis_index("s")
    tid = cid * _NS + sid

    def _init(i, _):
        ones[pl.ds(i * 16, 16)] = jnp.ones((16,), jnp.float32)
        zbuf[pl.ds(i * 16, 16)] = jnp.zeros((16,), jnp.float32)
        return 0

    lax.fori_loop(0, _CHUNK // 16, _init, 0)

    def _zinit(i, _):
        zbuf[pl.ds(i * 16, 16)] = jnp.zeros((16,), jnp.float32)
        return 0

    lax.fori_loop(0, _RPT // 16, _zinit, 0)

    pltpu.sync_copy(zbuf, deg_sh.at[pl.ds(sid * _RPT, _RPT)])
    pltpu.sync_copy(dst_hbm.at[pl.ds(tid * _CPT, _CPT)], didx)
    plsc.subcore_barrier()

    def _scat(j, _):
        pltpu.sync_copy(ones, deg_sh.at[didx.at[j]], add=True)
        return 0

    lax.fori_loop(0, _CPT, _scat, 0)
    plsc.subcore_barrier()

    pltpu.sync_copy(deg_sh.at[pl.ds(sid * _RPT, _RPT)], zbuf)
    pltpu.sync_copy(zbuf, out_hbm.at[cid, pl.ds(sid * _RPT, _RPT)])


@functools.partial(
    pl.kernel,
    out_type=jax.ShapeDtypeStruct((_NC, _NPAD, HID), jnp.float32),
    mesh=_mesh,
    compiler_params=pltpu.CompilerParams(use_tc_tiling_on_sc=False),
    scratch_types=[
        pltpu.VMEM((_CMAX, _CHUNK), jnp.int32),
        pltpu.VMEM((_CMAX, _CHUNK), jnp.int32),
        pltpu.VMEM((_D, _CHUNK, HID), jnp.float32),
        pltpu.VMEM((_RPT, HID), jnp.float32),
        pltpu.VMEM_SHARED((_NPAD, HID), jnp.float32),
        pltpu.SemaphoreType.DMA((_D,)),
        pltpu.SemaphoreType.DMA((_D,)),
    ],
)
def _agg_kernel(src_hbm, dst_hbm, g_hbm, out_hbm, sidx, didx, rows, zbuf, agg_sh,
                gsem, ssem):
    cid = lax.axis_index("c")
    sid = lax.axis_index("s")
    my_cnt = jnp.where(cid == 0, _C0, _C1)
    base_row = jnp.where(cid == 0, sid * _C0, _NS * _C0 + sid * _C1)

    def _zinit(i, _):
        zbuf[i, :] = jnp.zeros((HID,), jnp.float32)
        return 0

    lax.fori_loop(0, _RPT, _zinit, 0)

    pltpu.sync_copy(zbuf, agg_sh.at[pl.ds(sid * _RPT, _RPT)])

    @pl.when(cid == 0)
    def _():
        pltpu.sync_copy(src_hbm.at[pl.ds(base_row, _C0)], sidx.at[pl.ds(0, _C0)])
        pltpu.sync_copy(dst_hbm.at[pl.ds(base_row, _C0)], didx.at[pl.ds(0, _C0)])

    if _C1 > 0:
        @pl.when(cid == 1)
        def _():
            pltpu.sync_copy(src_hbm.at[pl.ds(base_row, _C1)], sidx.at[pl.ds(0, _C1)])
            pltpu.sync_copy(dst_hbm.at[pl.ds(base_row, _C1)], didx.at[pl.ds(0, _C1)])

    plsc.subcore_barrier()

    def _group(g, _):
        base = g * _D
        for b in range(_D):
            j = base + b

            @pl.when(g > 0)
            def _():
                pltpu.make_async_copy(
                    rows.at[b], agg_sh.at[didx.at[j]], ssem.at[b]
                ).wait()

            pltpu.async_copy(g_hbm.at[sidx.at[j]], rows.at[b], gsem.at[b])
        for b in range(_D):
            j = base + b
            pltpu.make_async_copy(g_hbm.at[sidx.at[j]], rows.at[b], gsem.at[b]).wait()
            pltpu.make_async_copy(rows.at[b], agg_sh.at[didx.at[j]], ssem.at[b]).start(
                add=True
            )
        return 0

    lax.fori_loop(0, my_cnt // _D, _group, 0)
    for b in range(_D):
        pltpu.make_async_copy(rows.at[b], agg_sh.at[didx.at[b]], ssem.at[b]).wait()
    plsc.subcore_barrier()

    pltpu.sync_copy(agg_sh.at[pl.ds(sid * _RPT, _RPT)], zbuf)
    pltpu.sync_copy(zbuf, out_hbm.at[cid, pl.ds(sid * _RPT, _RPT)])


_BLK = 1000
_GRID = N_NODES // _BLK


def _tc1_body(x_ref, w1_ref, degp_ref, g1_ref, dinv_ref):
    deg = 1.0 + degp_ref[:, 0] + degp_ref[:, 1]
    dinv = lax.rsqrt(deg)[:, None]
    h = jnp.dot(x_ref[...], w1_ref[...], preferred_element_type=jnp.float32)
    g1_ref[...] = h * dinv
    dinv_ref[...] = dinv


def _tc1(x, w1, degp):
    return pl.pallas_call(
        _tc1_body,
        grid=(_GRID,),
        in_specs=[
            pl.BlockSpec((_BLK, IN_CH), lambda i: (i, 0)),
            pl.BlockSpec((IN_CH, HID), lambda i: (0, 0)),
            pl.BlockSpec((_BLK, 2), lambda i: (i, 0)),
        ],
        out_specs=[
            pl.BlockSpec((_BLK, HID), lambda i: (i, 0)),
            pl.BlockSpec((_BLK, 1), lambda i: (i, 0)),
        ],
        out_shape=[
            jax.ShapeDtypeStruct((N_NODES, HID), jnp.float32),
            jax.ShapeDtypeStruct((N_NODES, 1), jnp.float32),
        ],
    )(x, w1, degp)


def _tc2_body(aggp_ref, g1_ref, dinv_ref, b1_ref, g2_ref):
    agg = aggp_ref[0] + aggp_ref[1] + g1_ref[...]
    dinv = dinv_ref[...]
    h1 = jnp.maximum(dinv * agg + b1_ref[...], 0.0)
    g2_ref[...] = dinv * h1


def _tc2(aggp, g1, dinv, b1):
    return pl.pallas_call(
        _tc2_body,
        grid=(_GRID,),
        in_specs=[
            pl.BlockSpec((2, _BLK, HID), lambda i: (0, i, 0)),
            pl.BlockSpec((_BLK, HID), lambda i: (i, 0)),
            pl.BlockSpec((_BLK, 1), lambda i: (i, 0)),
            pl.BlockSpec((1, HID), lambda i: (0, 0)),
        ],
        out_specs=pl.BlockSpec((_BLK, HID), lambda i: (i, 0)),
        out_shape=jax.ShapeDtypeStruct((N_NODES, HID), jnp.float32),
    )(aggp, g1, dinv, b1)


def _tc3_body(aggp_ref, g2_ref, dinv_ref, w2_ref, b2_ref, out_ref):
    p = dinv_ref[...] * (aggp_ref[0] + aggp_ref[1] + g2_ref[...])
    out_ref[...] = (
        jnp.dot(p, w2_ref[...], preferred_element_type=jnp.float32) + b2_ref[...]
    )


def _tc3(aggp, g2, dinv, w2, b2):
    return pl.pallas_call(
        _tc3_body,
        grid=(_GRID,),
        in_specs=[
            pl.BlockSpec((2, _BLK, HID), lambda i: (0, i, 0)),
            pl.BlockSpec((_BLK, HID), lambda i: (i, 0)),
            pl.BlockSpec((_BLK, 1), lambda i: (i, 0)),
            pl.BlockSpec((HID, OUT_CH), lambda i: (0, 0)),
            pl.BlockSpec((1, OUT_CH), lambda i: (0, 0)),
        ],
        out_specs=pl.BlockSpec((_BLK, OUT_CH), lambda i: (i, 0)),
        out_shape=jax.ShapeDtypeStruct((N_NODES, OUT_CH), jnp.float32),
    )(aggp, g2, dinv, w2, b2)


def kernel(x, edge_index, W1, b1, W2, b2):
    src = edge_index[0].astype(jnp.int32)
    dst = edge_index[1].astype(jnp.int32)
    pad = _E_PAD - N_EDGES
    src_r = jnp.concatenate([src, jnp.zeros((pad,), jnp.int32)]).reshape(_ROWS, _CHUNK)
    trash = _TRASH + jnp.arange(pad, dtype=jnp.int32) % (_NPAD - _TRASH)
    dst_r = jnp.concatenate([dst, trash]).reshape(_ROWS, _CHUNK)
    degp = _deg_kernel(dst_r)
    g1, dinv = _tc1(x, W1, degp.T)
    agg1 = _agg_kernel(src_r, dst_r, g1)
    g2 = _tc2(agg1, g1, dinv, b1.reshape(1, HID))
    agg2 = _agg_kernel(src_r, dst_r, g2)
    return _tc3(agg2, g2, dinv, W2, b2.reshape(1, OUT_CH))
